# baseline (speedup 1.0000x reference)
import jax
import jax.numpy as jnp
from jax import lax
from jax.experimental import pallas as pl
from jax.experimental.pallas import tpu as pltpu
from jax.experimental.pallas import tpu_sc as plsc

K = 32
BATCH = 16384
NC = 2
NS = 16
NW = NC * NS
BPW = BATCH // NW
L = 16
SB = 8
NBAT = BPW // SB
GROUPS = BPW // L


def _sc_body(uidx_hbm, vidx_hbm, w3_hbm, h3_hbm, dot_hbm, emb_hbm,
             idx_u, idx_v, bufs, cat, dvec, sems, sem_out):
    wid = lax.axis_index("s") * NC + lax.axis_index("c")
    base = wid * BPW

    pltpu.sync_copy(uidx_hbm.at[pl.ds(base, BPW)], idx_u)
    pltpu.sync_copy(vidx_hbm.at[pl.ds(base, BPW)], idx_v)

    def fire(uv, vv, lane0, p):
        for j in range(SB):
            u = uv[lane0 + j] >> 3
            v = vv[lane0 + j] >> 3
            pltpu.async_copy(
                w3_hbm.at[u], bufs.at[p, 0, pl.ds(j * 8, 8)], sems.at[p, 0])
            pltpu.async_copy(
                h3_hbm.at[v], bufs.at[p, 1, pl.ds(j * 8, 8)], sems.at[p, 1])

    def drain(p):
        pltpu.make_async_copy(
            w3_hbm.at[pl.ds(0, SB)], bufs.at[p, 0], sems.at[p, 0]).wait()
        pltpu.make_async_copy(
            h3_hbm.at[pl.ds(0, SB)], bufs.at[p, 1], sems.at[p, 1]).wait()

    def process(n, uv, vv, lane0, p):
        for j in range(SB):
            r = n * SB + j
            s = j * 8 + (uv[lane0 + j] & 7)
            t = j * 8 + (vv[lane0 + j] & 7)
            for half in range(2):
                cat[r, pl.ds(half * L, L)] = bufs[p, 0, s, pl.ds(half * L, L)]
                cat[r, pl.ds(K + half * L, L)] = bufs[p, 1, t, pl.ds(half * L, L)]

    uv0 = idx_u[pl.ds(0, L)]
    vv0 = idx_v[pl.ds(0, L)]
    fire(uv0, vv0, 0, 0)

    def body(it, carry):
        uv = idx_u[pl.ds(it * L, L)]
        vv = idx_v[pl.ds(it * L, L)]
        fire(uv, vv, SB, 1)
        drain(0)
        process(2 * it, uv, vv, 0, 0)

        @pl.when(2 * it + 2 < NBAT)
        def _():
            uvn = idx_u[pl.ds((it + 1) * L, L)]
            vvn = idx_v[pl.ds((it + 1) * L, L)]
            fire(uvn, vvn, 0, 0)

        drain(1)
        process(2 * it + 1, uv, vv, SB, 1)
        return carry

    lax.fori_loop(0, NBAT // 2, body, 0)

    iota16 = lax.iota(jnp.int32, L)

    def grp(g, carry):
        rows = g * L + iota16
        acc = jnp.zeros((L,), jnp.float32)
        for k in range(K):
            kv = jnp.full((L,), k, jnp.int32)
            kv2 = jnp.full((L,), K + k, jnp.int32)
            u = plsc.load_gather(cat, [rows, kv])
            v = plsc.load_gather(cat, [rows, kv2])
            acc = acc + u * v
        dvec[pl.ds(g * L, L)] = acc
        return carry

    lax.fori_loop(0, GROUPS, grp, 0)

    pltpu.async_copy(cat, emb_hbm.at[pl.ds(base, BPW)], sem_out).wait()
    pltpu.sync_copy(dvec, dot_hbm.at[pl.ds(base, BPW)])


@jax.jit
def _mf_dr(uidx, vidx, w3, h3):
    mesh = plsc.VectorSubcoreMesh(core_axis_name="c", subcore_axis_name="s")
    return pl.kernel(
        _sc_body,
        out_type=(
            jax.ShapeDtypeStruct((BATCH,), jnp.float32),
            jax.ShapeDtypeStruct((BATCH, 2 * K), jnp.float32),
        ),
        mesh=mesh,
        compiler_params=pltpu.CompilerParams(
            use_tc_tiling_on_sc=False, needs_layout_passes=False),
        scratch_types=[
            pltpu.VMEM((BPW,), jnp.int32),
            pltpu.VMEM((BPW,), jnp.int32),
            pltpu.VMEM((2, 2, SB * 8, K), jnp.float32),
            pltpu.VMEM((BPW, 2 * K), jnp.float32),
            pltpu.VMEM((BPW,), jnp.float32),
            pltpu.SemaphoreType.DMA((2, 2)),
            pltpu.SemaphoreType.DMA,
        ],
        name="mf_dr_sc",
    )(uidx, vidx, w3, h3)


def kernel(x, W, H):
    uidx = x[:, 0].astype(jnp.int32)
    vidx = x[:, 1].astype(jnp.int32)
    w3 = W.reshape(W.shape[0] // 8, 8, K)
    h3 = H.reshape(H.shape[0] // 8, 8, K)
    out, emb = _mf_dr(uidx, vidx, w3, h3)
    return (out, emb)


# speedup vs baseline: 2.2549x; 2.2549x over previous
import jax
import jax.numpy as jnp
from jax import lax
from jax.experimental import pallas as pl
from jax.experimental.pallas import tpu as pltpu
from jax.experimental.pallas import tpu_sc as plsc

K = 32
BATCH = 16384
NC = 2
NS = 16
NW = NC * NS
BPW = BATCH // NW
L = 16
SB = 8
NBAT = BPW // SB
GROUPS = BPW // L


def _sc_body(uidx_hbm, vidx_hbm, w3_hbm, h3_hbm, dot_hbm, emb_hbm,
             idx_u, idx_v, bufs, cat, dvec, sems, sem_out):
    wid = lax.axis_index("s") * NC + lax.axis_index("c")
    base = wid * BPW

    pltpu.sync_copy(uidx_hbm.at[pl.ds(base, BPW)], idx_u)
    pltpu.sync_copy(vidx_hbm.at[pl.ds(base, BPW)], idx_v)

    def fire(uv, vv, lane0, p):
        for j in range(SB):
            u = uv[lane0 + j] >> 3
            v = vv[lane0 + j] >> 3
            pltpu.async_copy(
                w3_hbm.at[u], bufs.at[p, 0, pl.ds(j * 8, 8)], sems.at[p, 0])
            pltpu.async_copy(
                h3_hbm.at[v], bufs.at[p, 1, pl.ds(j * 8, 8)], sems.at[p, 1])

    def drain(p):
        pltpu.make_async_copy(
            w3_hbm.at[pl.ds(0, SB)], bufs.at[p, 0], sems.at[p, 0]).wait()
        pltpu.make_async_copy(
            h3_hbm.at[pl.ds(0, SB)], bufs.at[p, 1], sems.at[p, 1]).wait()

    def process(n, uv, vv, lane0, p):
        for j in range(SB):
            r = n * SB + j
            s = j * 8 + (uv[lane0 + j] & 7)
            t = j * 8 + (vv[lane0 + j] & 7)
            for half in range(2):
                cat[r, pl.ds(half * L, L)] = bufs[p, 0, s, pl.ds(half * L, L)]
                cat[r, pl.ds(K + half * L, L)] = bufs[p, 1, t, pl.ds(half * L, L)]

    uv0 = idx_u[pl.ds(0, L)]
    vv0 = idx_v[pl.ds(0, L)]
    fire(uv0, vv0, 0, 0)

    def body(it, carry):
        uv = idx_u[pl.ds(it * L, L)]
        vv = idx_v[pl.ds(it * L, L)]
        fire(uv, vv, SB, 1)
        drain(0)
        process(2 * it, uv, vv, 0, 0)

        @pl.when(2 * it + 2 < NBAT)
        def _():
            uvn = idx_u[pl.ds((it + 1) * L, L)]
            vvn = idx_v[pl.ds((it + 1) * L, L)]
            fire(uvn, vvn, 0, 0)

        drain(1)
        process(2 * it + 1, uv, vv, SB, 1)
        return carry

    lax.fori_loop(0, NBAT // 2, body, 0)

    iota16 = lax.iota(jnp.int32, L)

    def grp(g, carry):
        rows = g * L + iota16
        acc = jnp.zeros((L,), jnp.float32)
        for k in range(K):
            kv = jnp.full((L,), k, jnp.int32)
            kv2 = jnp.full((L,), K + k, jnp.int32)
            u = plsc.load_gather(cat, [rows, kv])
            v = plsc.load_gather(cat, [rows, kv2])
            acc = acc + u * v
        dvec[pl.ds(g * L, L)] = acc
        return carry

    lax.fori_loop(0, GROUPS, grp, 0)

    pltpu.async_copy(cat, emb_hbm.at[pl.ds(base, BPW)], sem_out).wait()
    pltpu.sync_copy(dvec, dot_hbm.at[pl.ds(base, BPW)])


@jax.jit
def _mf_dr(uidx, vidx, w3, h3):
    mesh = plsc.VectorSubcoreMesh(core_axis_name="c", subcore_axis_name="s")
    return pl.kernel(
        _sc_body,
        out_type=(
            jax.ShapeDtypeStruct((BATCH,), jnp.float32),
            jax.ShapeDtypeStruct((BATCH, 2 * K), jnp.float32),
        ),
        mesh=mesh,
        compiler_params=pltpu.CompilerParams(needs_layout_passes=False),
        scratch_types=[
            pltpu.VMEM((BPW,), jnp.int32),
            pltpu.VMEM((BPW,), jnp.int32),
            pltpu.VMEM((2, 2, SB * 8, K), jnp.float32),
            pltpu.VMEM((BPW, 2 * K), jnp.float32),
            pltpu.VMEM((BPW,), jnp.float32),
            pltpu.SemaphoreType.DMA((2, 2)),
            pltpu.SemaphoreType.DMA,
        ],
        name="mf_dr_sc",
    )(uidx, vidx, w3, h3)


def kernel(x, W, H):
    uidx = x[:, 0].astype(jnp.int32)
    vidx = x[:, 1].astype(jnp.int32)
    w3 = W.reshape(W.shape[0] // 8, 8, K)
    h3 = H.reshape(H.shape[0] // 8, 8, K)
    out, emb = _mf_dr(uidx, vidx, w3, h3)
    return (out, emb)
